# baseline (speedup 1.0000x reference)
import functools

import jax
import jax.numpy as jnp
from jax import lax
from jax.experimental import pallas as pl
from jax.experimental.pallas import tpu as pltpu
from jax.experimental.pallas import tpu_sc as plsc

NX, NY = 2048, 2048
N = NX * NY
K = 4
NC, NS = 2, 16
NW = NC * NS
OW = N // NW
CHO = 4096
CH4 = CHO * K
NCH = OW // CHO
LANES = 16
RED_UNROLL = 4


@functools.partial(
    pl.kernel,
    out_type=jax.ShapeDtypeStruct((N,), jnp.float32),
    mesh=plsc.VectorSubcoreMesh(core_axis_name="c", subcore_axis_name="s"),
    scratch_types=[
        pltpu.VMEM((CH4,), jnp.int32),
        pltpu.VMEM((CH4,), jnp.int32),
        pltpu.VMEM((CH4,), jnp.float32),
        pltpu.VMEM((CH4,), jnp.float32),
        pltpu.VMEM((CH4,), jnp.float32),
        pltpu.VMEM((CH4,), jnp.float32),
        pltpu.VMEM((CHO,), jnp.float32),
        pltpu.VMEM((CHO,), jnp.float32),
        pltpu.SemaphoreType.DMA,
        pltpu.SemaphoreType.DMA,
        pltpu.SemaphoreType.DMA,
        pltpu.SemaphoreType.DMA,
        pltpu.SemaphoreType.DMA,
        pltpu.SemaphoreType.DMA,
    ],
)
def _bilinear_sc(f_hbm, lin_hbm, w_hbm, out_hbm,
                 idx0, idx1, vals0, vals1, w0, w1, outv0, outv1,
                 si0, si1, sg0, sg1, so0, so1):
    wid = lax.axis_index("s") * NC + lax.axis_index("c")
    obase = wid * OW
    idx_v = (idx0, idx1)
    vals_v = (vals0, vals1)
    w_v = (w0, w1)
    out_v = (outv0, outv1)
    sin = (si0, si1)
    sg = (sg0, sg1)
    so = (so0, so1)

    def fire_in(i):
        b = i % 2
        copies = []
        for k in range(K):
            src = pl.ds(k * N + obase + i * CHO, CHO)
            dst = pl.ds(k * CHO, CHO)
            copies.append(
                pltpu.async_copy(lin_hbm.at[src], idx_v[b].at[dst], sin[b]))
            copies.append(
                pltpu.async_copy(w_hbm.at[src], w_v[b].at[dst], sin[b]))
        return tuple(copies)

    def fire_gather(i, pend):
        b = i % 2
        for c in pend[i]:
            c.wait()
        pend[i] = ()
        h = CH4 // 2
        return (
            pltpu.async_copy(f_hbm.at[idx_v[b].at[pl.ds(0, h)]],
                             vals_v[b].at[pl.ds(0, h)], sg[b]),
            pltpu.async_copy(f_hbm.at[idx_v[b].at[pl.ds(h, h)]],
                             vals_v[b].at[pl.ds(h, h)], sg[b]),
        )

    pend = {}
    pend[0] = fire_in(0)
    pend[1] = fire_in(1)
    gathers = {0: fire_gather(0, pend)}
    outs = {}

    for i in range(NCH):
        b = i % 2
        if i + 1 < NCH:
            gathers[i + 1] = fire_gather(i + 1, pend)
        for g in gathers.pop(i):
            g.wait()
        if i >= 2:
            outs.pop(i).wait()

        def red(j, _, b=b):
            jbase = j * (LANES * RED_UNROLL)
            for u in range(RED_UNROLL):
                acc = None
                for k in range(K):
                    s = pl.ds(k * CHO + u * LANES + jbase, LANES)
                    p = vals_v[b][s] * w_v[b][s]
                    acc = p if acc is None else acc + p
                out_v[b][pl.ds(u * LANES + jbase, LANES)] = acc
            return 0

        lax.fori_loop(0, CHO // (LANES * RED_UNROLL), red, 0)

        outs[i + 2] = pltpu.async_copy(
            out_v[b], out_hbm.at[pl.ds(obase + i * CHO, CHO)], so[b])
        if i + 2 < NCH:
            pend[i + 2] = fire_in(i + 2)

    outs.pop(NCH).wait()
    outs.pop(NCH + 1).wait()


def kernel(f_plane, ix, iy, w, dl):
    nx, ny = f_plane.shape
    lin = ix.astype(jnp.int32) * ny + iy.astype(jnp.int32)
    lin_t = jnp.transpose(lin, (2, 0, 1)).reshape(-1)
    w_t = jnp.transpose(w, (2, 0, 1)).reshape(-1)
    out = _bilinear_sc(f_plane.reshape(-1), lin_t, w_t)
    return out.reshape(nx, ny)


# speedup vs baseline: 1.2796x; 1.2796x over previous
import functools

import jax
import jax.numpy as jnp
from jax import lax
from jax.experimental import pallas as pl
from jax.experimental.pallas import tpu as pltpu
from jax.experimental.pallas import tpu_sc as plsc

NX, NY = 2048, 2048
N = NX * NY
K = 4
NC, NS = 2, 16
NW = NC * NS
OW = N // NW
CHO = 4096
CH4 = CHO * K
NCH = OW // CHO
LANES = 16
RED_UNROLL = 4


@functools.partial(
    pl.kernel,
    out_type=jax.ShapeDtypeStruct((N,), jnp.float32),
    mesh=plsc.VectorSubcoreMesh(core_axis_name="c", subcore_axis_name="s"),
    scratch_types=[
        pltpu.VMEM((CH4,), jnp.int32),
        pltpu.VMEM((CH4,), jnp.int32),
        pltpu.VMEM((CH4,), jnp.float32),
        pltpu.VMEM((CH4,), jnp.float32),
        pltpu.VMEM((CH4,), jnp.float32),
        pltpu.VMEM((CH4,), jnp.float32),
        pltpu.VMEM((CHO,), jnp.float32),
        pltpu.VMEM((CHO,), jnp.float32),
        pltpu.SemaphoreType.DMA,
        pltpu.SemaphoreType.DMA,
        pltpu.SemaphoreType.DMA,
        pltpu.SemaphoreType.DMA,
        pltpu.SemaphoreType.DMA,
        pltpu.SemaphoreType.DMA,
    ],
)
def _bilinear_sc(f_hbm, lin_hbm, w_hbm, out_hbm,
                 idx0, idx1, vals0, vals1, w0, w1, outv0, outv1,
                 si0, si1, sg0, sg1, so0, so1):
    wid = lax.axis_index("s") * NC + lax.axis_index("c")
    obase = wid * OW
    idx_v = (idx0, idx1)
    vals_v = (vals0, vals1)
    w_v = (w0, w1)
    out_v = (outv0, outv1)
    sin = (si0, si1)
    sg = (sg0, sg1)
    so = (so0, so1)

    def fire_in(i):
        b = i % 2
        src = pl.ds((obase + i * CHO) * K, CH4)
        return (
            pltpu.async_copy(lin_hbm.at[src], idx_v[b], sin[b]),
            pltpu.async_copy(w_hbm.at[src], w_v[b], sin[b]),
        )

    def fire_gather(i, pend):
        b = i % 2
        for c in pend[i]:
            c.wait()
        pend[i] = ()
        return pltpu.async_copy(f_hbm.at[idx_v[b]], vals_v[b], sg[b])

    pend = {}
    pend[0] = fire_in(0)
    pend[1] = fire_in(1)
    gathers = {0: fire_gather(0, pend)}
    outs = {}

    for i in range(NCH):
        b = i % 2
        if i + 1 < NCH:
            gathers[i + 1] = fire_gather(i + 1, pend)
        gathers.pop(i).wait()
        if i >= 2:
            outs.pop(i).wait()

        def red(t, _, b=b):
            tbase = t * (LANES * 32)
            for u in range(8):
                acc = None
                for k in range(K):
                    s = pl.ds(tbase + k * 128 + u * LANES, LANES)
                    p = vals_v[b][s] * w_v[b][s]
                    acc = p if acc is None else acc + p
                out_v[b][pl.ds(t * 128 + u * LANES, LANES)] = acc
            return 0

        lax.fori_loop(0, CHO // 128, red, 0)

        outs[i + 2] = pltpu.async_copy(
            out_v[b], out_hbm.at[pl.ds(obase + i * CHO, CHO)], so[b])
        if i + 2 < NCH:
            pend[i + 2] = fire_in(i + 2)

    outs.pop(NCH).wait()
    outs.pop(NCH + 1).wait()


def _tile_order(x, nx, ny):
    return x.reshape(nx, ny // 128, 128, K).transpose(0, 1, 3, 2).reshape(-1)


def kernel(f_plane, ix, iy, w, dl):
    nx, ny = f_plane.shape
    lin = ix.astype(jnp.int32) * ny + iy.astype(jnp.int32)
    lin_t = _tile_order(lin, nx, ny)
    w_t = _tile_order(w, nx, ny)
    out = _bilinear_sc(f_plane.reshape(-1), lin_t, w_t)
    return out.reshape(nx, ny)


# speedup vs baseline: 1.3018x; 1.0174x over previous
import functools

import jax
import jax.numpy as jnp
from jax import lax
from jax.experimental import pallas as pl
from jax.experimental.pallas import tpu as pltpu
from jax.experimental.pallas import tpu_sc as plsc

NX, NY = 2048, 2048
N = NX * NY
K = 4
NC, NS = 2, 16
NW = NC * NS
OW = N // NW
CHO = 4096
CH4 = CHO * K
NCH = OW // CHO
LANES = 16
RED_UNROLL = 4


@functools.partial(
    pl.kernel,
    out_type=jax.ShapeDtypeStruct((N,), jnp.float32),
    mesh=plsc.VectorSubcoreMesh(core_axis_name="c", subcore_axis_name="s"),
    scratch_types=[
        pltpu.VMEM((CH4,), jnp.int32),
        pltpu.VMEM((CH4,), jnp.int32),
        pltpu.VMEM((CH4,), jnp.float32),
        pltpu.VMEM((CH4,), jnp.float32),
        pltpu.VMEM((CH4,), jnp.float32),
        pltpu.VMEM((CH4,), jnp.float32),
        pltpu.VMEM((CHO,), jnp.float32),
        pltpu.VMEM((CHO,), jnp.float32),
        pltpu.SemaphoreType.DMA,
        pltpu.SemaphoreType.DMA,
        pltpu.SemaphoreType.DMA,
        pltpu.SemaphoreType.DMA,
        pltpu.SemaphoreType.DMA,
        pltpu.SemaphoreType.DMA,
    ],
)
def _bilinear_sc(f_hbm, lin_hbm, w_hbm, out_hbm,
                 idx0, idx1, vals0, vals1, w0, w1, outv0, outv1,
                 si0, si1, sg0, sg1, so0, so1):
    wid = lax.axis_index("s") * NC + lax.axis_index("c")
    obase = wid * OW
    idx_v = (idx0, idx1)
    vals_v = (vals0, vals1)
    w_v = (w0, w1)
    out_v = (outv0, outv1)
    sin = (si0, si1)
    sg = (sg0, sg1)
    so = (so0, so1)

    def fire_in(i):
        b = i % 2
        src = pl.ds((obase + i * CHO) * K, CH4)
        return (
            pltpu.async_copy(lin_hbm.at[src], idx_v[b], sin[b]),
            pltpu.async_copy(w_hbm.at[src], w_v[b], sin[b]),
        )

    def fire_gather(i, pend):
        b = i % 2
        for c in pend[i]:
            c.wait()
        pend[i] = ()
        return pltpu.async_copy(f_hbm.at[idx_v[b]], vals_v[b], sg[b])

    pend = {}
    pend[0] = fire_in(0)
    pend[1] = fire_in(1)
    gathers = {0: fire_gather(0, pend)}
    outs = {}

    for i in range(NCH):
        b = i % 2
        if i + 1 < NCH:
            gathers[i + 1] = fire_gather(i + 1, pend)
        gathers.pop(i).wait()
        if i >= 2:
            outs.pop(i).wait()

        def red(t, _, b=b):
            tbase = t * (LANES * 32)
            for u in range(8):
                acc = None
                for k in range(K):
                    s = pl.ds(tbase + k * 128 + u * LANES, LANES)
                    p = vals_v[b][s] * w_v[b][s]
                    acc = p if acc is None else acc + p
                out_v[b][pl.ds(t * 128 + u * LANES, LANES)] = acc
            return 0

        lax.fori_loop(0, CHO // 128, red, 0)

        outs[i + 2] = pltpu.async_copy(
            out_v[b], out_hbm.at[pl.ds(obase + i * CHO, CHO)], so[b])
        if i + 2 < NCH:
            pend[i + 2] = fire_in(i + 2)

    outs.pop(NCH).wait()
    outs.pop(NCH + 1).wait()


def _tile_order(x, nx, ny):
    return x.reshape(nx, ny // 128, 128, K).transpose(0, 1, 3, 2).reshape(-1)


def kernel(f_plane, ix, iy, w, dl):
    nx, ny = f_plane.shape
    ixi = ix.astype(jnp.int32)
    iyi = iy.astype(jnp.int32)
    lin = ((((ixi >> 3) << 4) + (iyi >> 7)) << 10) + ((ixi & 7) << 7) + (iyi & 127)
    lin_t = _tile_order(lin, nx, ny)
    w_t = _tile_order(w, nx, ny)
    f_t = f_plane.reshape(nx // 8, 8, ny // 128, 128)
    f_t = f_t.transpose(0, 2, 1, 3).reshape(-1)
    out = _bilinear_sc(f_t, lin_t, w_t)
    return out.reshape(nx, ny)


# speedup vs baseline: 1.3266x; 1.0191x over previous
import functools

import jax
import jax.numpy as jnp
from jax import lax
from jax.experimental import pallas as pl
from jax.experimental.pallas import tpu as pltpu
from jax.experimental.pallas import tpu_sc as plsc

NX, NY = 2048, 2048
N = NX * NY
K = 4
NC, NS = 2, 16
NW = NC * NS
OW = N // NW
CHO = 4096
CH4 = CHO * K
NCH = OW // CHO
LANES = 16
RED_UNROLL = 4


@functools.partial(
    pl.kernel,
    out_type=jax.ShapeDtypeStruct((N,), jnp.float32),
    mesh=plsc.VectorSubcoreMesh(core_axis_name="c", subcore_axis_name="s"),
    compiler_params=pltpu.CompilerParams(needs_layout_passes=False),
    scratch_types=[
        pltpu.VMEM((CH4,), jnp.int32),
        pltpu.VMEM((CH4,), jnp.int32),
        pltpu.VMEM((CH4,), jnp.int32),
        pltpu.VMEM((CH4,), jnp.int32),
        pltpu.VMEM((CH4,), jnp.float32),
        pltpu.VMEM((CH4,), jnp.float32),
        pltpu.VMEM((CHO,), jnp.float32),
        pltpu.VMEM((CHO,), jnp.float32),
        pltpu.SemaphoreType.DMA,
        pltpu.SemaphoreType.DMA,
        pltpu.SemaphoreType.DMA,
        pltpu.SemaphoreType.DMA,
        pltpu.SemaphoreType.DMA,
        pltpu.SemaphoreType.DMA,
    ],
)
def _bilinear_sc(f_hbm, ix_hbm, iy_hbm, w_hbm, out_hbm,
                 idx0, idx1, vals0, vals1, w0, w1, outv0, outv1,
                 si0, si1, sg0, sg1, so0, so1):
    wid = lax.axis_index("s") * NC + lax.axis_index("c")
    obase = wid * OW
    idx_v = (idx0, idx1)
    vals_v = (vals0, vals1)
    w_v = (w0, w1)
    out_v = (outv0, outv1)
    sin = (si0, si1)
    sg = (sg0, sg1)
    so = (so0, so1)

    def fire_in(i):
        b = i % 2
        src = pl.ds((obase + i * CHO) * K, CH4)
        return (
            pltpu.async_copy(ix_hbm.at[src], idx_v[b], sin[b]),
            pltpu.async_copy(iy_hbm.at[src], vals_v[b], sin[b]),
            pltpu.async_copy(w_hbm.at[src], w_v[b], sin[b]),
        )

    def stage(i, pend):
        b = i % 2
        for c in pend[i]:
            c.wait()
        pend[i] = ()

        def lin_body(j, _, b=b):
            s = pl.ds(j * LANES, LANES)
            ixv = idx_v[b][s]
            iyv = vals_v[b][s]
            idx_v[b][s] = ((((ixv >> 3) << 4) + (iyv >> 7)) << 10) + (
                (ixv & 7) << 7) + (iyv & 127)
            return 0

        lax.fori_loop(0, CH4 // LANES, lin_body, 0)
        return pltpu.async_copy(f_hbm.at[idx_v[b]], vals_v[b], sg[b])

    pend = {}
    pend[0] = fire_in(0)
    pend[1] = fire_in(1)
    gathers = {0: stage(0, pend)}
    outs = {}

    for i in range(NCH):
        b = i % 2
        if i + 1 < NCH:
            gathers[i + 1] = stage(i + 1, pend)
        gathers.pop(i).wait()
        if i >= 2:
            outs.pop(i).wait()

        def red(q, _, b=b):
            t = q >> 3
            u = q & 7
            tbase = t * (LANES * 32) + u * LANES
            acc = None
            for k in range(K):
                s = pl.ds(tbase + k * 128, LANES)
                p = plsc.bitcast(vals_v[b][s], jnp.float32) * w_v[b][s]
                acc = p if acc is None else acc + p
            out_v[b][pl.ds(q * LANES, LANES)] = acc
            return 0

        lax.fori_loop(0, CHO // LANES, red, 0)

        outs[i + 2] = pltpu.async_copy(
            out_v[b], out_hbm.at[pl.ds(obase + i * CHO, CHO)], so[b])
        if i + 2 < NCH:
            pend[i + 2] = fire_in(i + 2)

    outs.pop(NCH).wait()
    outs.pop(NCH + 1).wait()


def _tile_order(x, nx, ny):
    return x.reshape(nx, ny // 128, 128, K).transpose(0, 1, 3, 2).reshape(-1)


def kernel(f_plane, ix, iy, w, dl):
    nx, ny = f_plane.shape
    ix_t = _tile_order(ix.astype(jnp.int32), nx, ny)
    iy_t = _tile_order(iy.astype(jnp.int32), nx, ny)
    w_t = _tile_order(w, nx, ny)
    f_i = jax.lax.bitcast_convert_type(f_plane, jnp.int32)
    f_t = f_i.reshape(nx // 8, 8, ny // 128, 128).transpose(0, 2, 1, 3).reshape(-1)
    out = _bilinear_sc(f_t, ix_t, iy_t, w_t)
    return out.reshape(nx, ny)


# speedup vs baseline: 1.3587x; 1.0242x over previous
import functools

import jax
import jax.numpy as jnp
from jax import lax
from jax.experimental import pallas as pl
from jax.experimental.pallas import tpu as pltpu
from jax.experimental.pallas import tpu_sc as plsc

NX, NY = 2048, 2048
N = NX * NY
K = 4
NC, NS = 2, 16
NW = NC * NS
OW = N // NW
CHO = 4096
CH4 = CHO * K
NCH = OW // CHO
LANES = 16
RED_UNROLL = 4


@functools.partial(
    pl.kernel,
    out_type=jax.ShapeDtypeStruct((N,), jnp.float32),
    mesh=plsc.VectorSubcoreMesh(core_axis_name="c", subcore_axis_name="s"),
    compiler_params=pltpu.CompilerParams(needs_layout_passes=False),
    scratch_types=[
        pltpu.VMEM((CH4,), jnp.int32),
        pltpu.VMEM((CH4,), jnp.int32),
        pltpu.VMEM((CH4,), jnp.int32),
        pltpu.VMEM((CH4,), jnp.int32),
        pltpu.VMEM((CH4,), jnp.float32),
        pltpu.VMEM((CH4,), jnp.float32),
        pltpu.VMEM((CHO,), jnp.float32),
        pltpu.VMEM((CHO,), jnp.float32),
        pltpu.SemaphoreType.DMA,
        pltpu.SemaphoreType.DMA,
        pltpu.SemaphoreType.DMA,
        pltpu.SemaphoreType.DMA,
        pltpu.SemaphoreType.DMA,
        pltpu.SemaphoreType.DMA,
    ],
)
def _bilinear_sc(f_hbm, ix_hbm, iy_hbm, w_hbm, out_hbm,
                 idx0, idx1, vals0, vals1, w0, w1, outv0, outv1,
                 si0, si1, sg0, sg1, so0, so1):
    wid = lax.axis_index("s") * NC + lax.axis_index("c")
    obase = wid * OW
    idx_v = (idx0, idx1)
    vals_v = (vals0, vals1)
    w_v = (w0, w1)
    out_v = (outv0, outv1)
    sin = (si0, si1)
    sg = (sg0, sg1)
    so = (so0, so1)

    def fire_in(i):
        b = i % 2
        tt0 = wid * (OW // 1024) + i * 4
        ti = tt0 // 16
        tj0 = tt0 % 16
        for p in range(8):
            src = pl.ds(((ti * 8 + p) * 16 + tj0) * 512, 2048)
            dst = pl.ds(p * 2048, 2048)
            pltpu.async_copy(ix_hbm.at[src], idx_v[b].at[dst], sin[b])
            pltpu.async_copy(iy_hbm.at[src], vals_v[b].at[dst], sin[b])
            pltpu.async_copy(w_hbm.at[src], w_v[b].at[dst], sin[b])
        full = pl.ds(0, CH4)
        return (
            pltpu.make_async_copy(ix_hbm.at[full], idx_v[b], sin[b]),
            pltpu.make_async_copy(iy_hbm.at[full], vals_v[b], sin[b]),
            pltpu.make_async_copy(w_hbm.at[full], w_v[b], sin[b]),
        )

    def stage(i, pend):
        b = i % 2
        for c in pend[i]:
            c.wait()
        pend[i] = ()

        def lin_body(j, _, b=b):
            s = pl.ds(j * LANES, LANES)
            ixv = idx_v[b][s]
            iyv = vals_v[b][s]
            idx_v[b][s] = ((((ixv >> 3) << 4) + (iyv >> 7)) << 10) + (
                (ixv & 7) << 7) + (iyv & 127)
            return 0

        lax.fori_loop(0, CH4 // LANES, lin_body, 0)
        return pltpu.async_copy(f_hbm.at[idx_v[b]], vals_v[b], sg[b])

    pend = {}
    pend[0] = fire_in(0)
    pend[1] = fire_in(1)
    gathers = {0: stage(0, pend)}
    outs = {}

    for i in range(NCH):
        b = i % 2
        if i + 1 < NCH:
            gathers[i + 1] = stage(i + 1, pend)
        gathers.pop(i).wait()
        if i >= 2:
            outs.pop(i).wait()

        def red(q, _, b=b):
            jw = q & 7
            ii = (q >> 3) & 7
            tt = q >> 6
            tbase = (ii << 11) + (tt << 9) + (jw << 4)
            acc = None
            for k in range(K):
                s = pl.ds(tbase + k * 128, LANES)
                p = plsc.bitcast(vals_v[b][s], jnp.float32) * w_v[b][s]
                acc = p if acc is None else acc + p
            out_v[b][pl.ds(q * LANES, LANES)] = acc
            return 0

        lax.fori_loop(0, CHO // LANES, red, 0)

        outs[i + 2] = pltpu.async_copy(
            out_v[b], out_hbm.at[pl.ds(obase + i * CHO, CHO)], so[b])
        if i + 2 < NCH:
            pend[i + 2] = fire_in(i + 2)

    outs.pop(NCH).wait()
    outs.pop(NCH + 1).wait()


def _tile_order(x, nx, ny):
    return x.reshape(nx, ny // 128, 128, K).transpose(0, 1, 3, 2).reshape(-1)


def kernel(f_plane, ix, iy, w, dl):
    nx, ny = f_plane.shape
    ix_t = _tile_order(ix.astype(jnp.int32), nx, ny)
    iy_t = _tile_order(iy.astype(jnp.int32), nx, ny)
    w_t = _tile_order(w, nx, ny)
    f_i = jax.lax.bitcast_convert_type(f_plane, jnp.int32)
    f_t = f_i.reshape(nx // 8, 8, ny // 128, 128).transpose(0, 2, 1, 3).reshape(-1)
    out = _bilinear_sc(f_t, ix_t, iy_t, w_t)
    out = out.reshape(nx // 8, ny // 128, 8, 128).transpose(0, 2, 1, 3)
    return out.reshape(nx, ny)
